# baseline (speedup 1.0000x reference)
import functools

import jax
import jax.numpy as jnp
from jax import lax
from jax.experimental import pallas as pl
from jax.experimental.pallas import tpu as pltpu
from jax.experimental.pallas import tpu_sc as plsc

N = 10000
E = 160000
D = 256

NW = 32
NPW = 313
NPAD = NW * NPW

CHUNK = 16000
NCHUNK = E // CHUNK
SEG = CHUNK
SEGW = NCHUNK * SEG
CROW = 256
G = 128

_MESH = plsc.VectorSubcoreMesh(core_axis_name="c", subcore_axis_name="s")
_SC_PARAMS = pltpu.CompilerParams(needs_layout_passes=False)


def _wid():
    return lax.axis_index("s") * 2 + lax.axis_index("c")


def _prep_body(src_hbm, dst_hbm, pair_hbm, cnt_hbm,
               dstv, srcv, stage_p, cbuf):
    w = _wid()
    lo = w * NPW
    hi = lo + NPW

    def chunk_body(ci, _):
        coff = pl.multiple_of(ci * CHUNK, 128)
        pltpu.sync_copy(dst_hbm.at[pl.ds(coff, CHUNK)], dstv)
        pltpu.sync_copy(src_hbm.at[pl.ds(coff, CHUNK)], srcv)

        def grp(g, fill):
            d16 = dstv[pl.ds(g * 16, 16)]
            s16 = srcv[pl.ds(g * 16, 16)]
            msk = (d16 >= lo) & (d16 < hi)
            ones = jnp.where(msk, jnp.int32(1), jnp.int32(0))
            csum = plsc.cumsum(ones)
            pos2 = 2 * (fill + csum - ones)
            plsc.store_scatter(stage_p, [pos2], s16, mask=msk)
            plsc.store_scatter(stage_p, [pos2 + 1], d16 - lo, mask=msk)
            return fill + csum[15]

        fill = lax.fori_loop(0, CHUNK // 16, grp, 0)
        cbuf[pl.ds(pl.multiple_of(ci * 16, 16), 16)] = (
            jnp.broadcast_to(fill, (16,)).astype(jnp.int32))
        soff = pl.multiple_of(w * 2 * SEGW + ci * 2 * SEG, 128)
        pltpu.sync_copy(stage_p, pair_hbm.at[pl.ds(soff, 2 * SEG)])
        return 0

    lax.fori_loop(0, NCHUNK, chunk_body, 0)
    pltpu.sync_copy(cbuf, cnt_hbm.at[pl.ds(pl.multiple_of(w * CROW, 128),
                                           CROW)])


_prep = pl.kernel(
    _prep_body,
    out_type=(
        jax.ShapeDtypeStruct((NW * 2 * SEGW,), jnp.int32),
        jax.ShapeDtypeStruct((NW * CROW,), jnp.int32),
    ),
    mesh=_MESH,
    scratch_types=[
        pltpu.VMEM((CHUNK,), jnp.int32),
        pltpu.VMEM((CHUNK,), jnp.int32),
        pltpu.VMEM((2 * SEG,), jnp.int32),
        pltpu.VMEM((CROW,), jnp.int32),
    ],
    compiler_params=_SC_PARAMS,
)


def _agg_body(m_hbm, pair_hbm, cnt_hbm, out_hbm,
              acc, rows, pbuf0, pbuf1, idxv, edlv,
              cntv, semg, semp0, semp1):
    w = _wid()
    pbuf = (pbuf0, pbuf1)
    semp = (semp0, semp1)

    def zero(i, _):
        acc[pl.ds(i * 16, 16)] = jnp.zeros((16,), jnp.float32)
        return 0

    lax.fori_loop(0, (NPW + 1) * D // 16, zero, 0)

    pltpu.sync_copy(cnt_hbm.at[pl.ds(pl.multiple_of(w * CROW, 128), CROW)],
                    cntv)

    def seg(ci, _):
        cnt = cntv[pl.ds(pl.multiple_of(ci * 16, 16), 16)][0]
        nblocks = (cnt + G - 1) // G
        segbase = w * 2 * SEGW + ci * 2 * SEG

        def pair_slice(bb):
            off = pl.multiple_of(segbase + bb * 2 * G, 128)
            return pair_hbm.at[pl.ds(off, 2 * G)]

        def startpair(bb, p):
            pltpu.async_copy(pair_slice(bb), pbuf[p], semp[p])

        def process(bb, p):
            pltpu.make_async_copy(pair_slice(bb), pbuf[p], semp[p]).wait()
            iota2 = lax.iota(jnp.int32, 16) * 2
            for j in range(G // 16):
                ev = plsc.load_gather(pbuf[p], [iota2 + (j * 32)])
                od = plsc.load_gather(pbuf[p], [iota2 + (j * 32 + 1)])
                idxv[pl.ds(j * 16, 16)] = jnp.clip(ev, 0, N - 1)
                edlv[pl.ds(j * 16, 16)] = od
            pltpu.async_copy(m_hbm.at[idxv], rows, semg).wait()
            lbase = bb * G
            ne = jnp.minimum(cnt - lbase, G)
            ne_v = jnp.broadcast_to(ne, (16,))
            ngroups = (ne + 15) // 16

            def grp(gi, _):
                gbase = pl.multiple_of(gi * 16, 16)
                dls = edlv[pl.ds(gbase, 16)]
                valid = (lax.iota(jnp.int32, 16) + gbase) < ne_v
                dls = jnp.where(valid, jnp.clip(dls, 0, NPW - 1), NPW)
                for k in range(16):
                    boff = dls[k] * D
                    erow = gbase + k
                    for v in range(D // 16):
                        sl = pl.ds(boff + v * 16, 16)
                        acc[sl] = jnp.maximum(
                            acc[sl], rows[erow, pl.ds(v * 16, 16)])
                return 0

            lax.fori_loop(0, ngroups, grp, 0)

        @pl.when(nblocks > 0)
        def _prologue():
            startpair(0, 0)

        def outer(ob, _):
            for p in (0, 1):
                bb = ob * 2 + p

                @pl.when(bb + 1 < nblocks)
                def _start_next():
                    startpair(bb + 1, 1 - p)

                @pl.when(bb < nblocks)
                def _process():
                    process(bb, p)
            return 0

        lax.fori_loop(0, (nblocks + 1) // 2, outer, 0)
        return 0

    lax.fori_loop(0, NCHUNK, seg, 0)
    pltpu.sync_copy(acc.at[pl.ds(0, NPW * D)],
                    out_hbm.at[pl.ds(pl.multiple_of(w * NPW * D, 128),
                                     NPW * D)])


_agg = pl.kernel(
    _agg_body,
    out_type=jax.ShapeDtypeStruct((NPAD * D,), jnp.float32),
    mesh=_MESH,
    scratch_types=[
        pltpu.VMEM(((NPW + 1) * D,), jnp.float32),
        pltpu.VMEM((G, D), jnp.float32),
        pltpu.VMEM((2 * G,), jnp.int32),
        pltpu.VMEM((2 * G,), jnp.int32),
        pltpu.VMEM((G,), jnp.int32),
        pltpu.VMEM((G + 16,), jnp.int32),
        pltpu.VMEM((CROW,), jnp.int32),
        pltpu.SemaphoreType.DMA,
        pltpu.SemaphoreType.DMA,
        pltpu.SemaphoreType.DMA,
    ],
    compiler_params=_SC_PARAMS,
)


RB = 1000
GRID = N // RB


def _l2norm(x):
    n = jnp.sqrt(jnp.sum(x * x, axis=1, keepdims=True))
    return x / jnp.maximum(n, 1e-12)


def _row_spec():
    return pl.BlockSpec((RB, D), lambda i: (i, 0))


def _full_spec():
    return pl.BlockSpec((D, D), lambda i: (0, 0))


def _bias_spec():
    return pl.BlockSpec((1, D), lambda i: (0, 0))


def _pool_body(h_ref, wp_ref, bp_ref, m_ref):
    m = jnp.dot(h_ref[...], wp_ref[...], preferred_element_type=jnp.float32)
    m_ref[...] = jnp.maximum(m + bp_ref[...], 0.0)


def _tc_pool(h, Wp, bp):
    return pl.pallas_call(
        _pool_body,
        grid=(GRID,),
        in_specs=[_row_spec(), _full_spec(), _bias_spec()],
        out_specs=_row_spec(),
        out_shape=jax.ShapeDtypeStruct((N, D), jnp.float32),
    )(h, Wp, bp.reshape(1, D))


def _mid_body(h_ref, ng_ref, ws_ref, wn_ref, b_ref, wp_ref, bp_ref,
              h1_ref, m1_ref):
    out = jnp.dot(h_ref[...], ws_ref[...], preferred_element_type=jnp.float32)
    out += jnp.dot(ng_ref[...], wn_ref[...], preferred_element_type=jnp.float32)
    out += b_ref[...]
    h1 = _l2norm(jnp.maximum(out, 0.0))
    h1_ref[...] = h1
    m1 = jnp.dot(h1, wp_ref[...], preferred_element_type=jnp.float32)
    m1_ref[...] = jnp.maximum(m1 + bp_ref[...], 0.0)


def _tc_mid(h, ng, Ws, Wn, b, Wp, bp):
    return pl.pallas_call(
        _mid_body,
        grid=(GRID,),
        in_specs=[_row_spec(), _row_spec(), _full_spec(), _full_spec(),
                  _bias_spec(), _full_spec(), _bias_spec()],
        out_specs=[_row_spec(), _row_spec()],
        out_shape=[jax.ShapeDtypeStruct((N, D), jnp.float32),
                   jax.ShapeDtypeStruct((N, D), jnp.float32)],
    )(h, ng, Ws, Wn, b.reshape(1, D), Wp, bp.reshape(1, D))


def _mid_enc_body(h_ref, ng_ref, ws_ref, wn_ref, b_ref, wp_ref, bp_ref,
                  enc_ref, h1_ref, m1_ref):
    out = jnp.dot(h_ref[...], ws_ref[...], preferred_element_type=jnp.float32)
    out += jnp.dot(ng_ref[...], wn_ref[...], preferred_element_type=jnp.float32)
    out += b_ref[...]
    enc_ref[...] = _l2norm(out)
    h1 = _l2norm(jnp.maximum(out, 0.0))
    h1_ref[...] = h1
    m1 = jnp.dot(h1, wp_ref[...], preferred_element_type=jnp.float32)
    m1_ref[...] = jnp.maximum(m1 + bp_ref[...], 0.0)


def _tc_mid_enc(h, ng, Ws, Wn, b, Wp, bp):
    return pl.pallas_call(
        _mid_enc_body,
        grid=(GRID,),
        in_specs=[_row_spec(), _row_spec(), _full_spec(), _full_spec(),
                  _bias_spec(), _full_spec(), _bias_spec()],
        out_specs=[_row_spec(), _row_spec(), _row_spec()],
        out_shape=[jax.ShapeDtypeStruct((N, D), jnp.float32),
                   jax.ShapeDtypeStruct((N, D), jnp.float32),
                   jax.ShapeDtypeStruct((N, D), jnp.float32)],
    )(h, ng, Ws, Wn, b.reshape(1, D), Wp, bp.reshape(1, D))


def _last_body(h_ref, ng_ref, ws_ref, wn_ref, b_ref, out_ref):
    out = jnp.dot(h_ref[...], ws_ref[...], preferred_element_type=jnp.float32)
    out += jnp.dot(ng_ref[...], wn_ref[...], preferred_element_type=jnp.float32)
    out_ref[...] = out + b_ref[...]


def _tc_last(h, ng, Ws, Wn, b):
    return pl.pallas_call(
        _last_body,
        grid=(GRID,),
        in_specs=[_row_spec(), _row_spec(), _full_spec(), _full_spec(),
                  _bias_spec()],
        out_specs=_row_spec(),
        out_shape=jax.ShapeDtypeStruct((N, D), jnp.float32),
    )(h, ng, Ws, Wn, b.reshape(1, D))


def _neigh(m, pair, cnts):
    flat = _agg(m, pair, cnts)
    return flat.reshape(NPAD, D)[:N]


def kernel(inputs, edge_index, Wp0, bp0, Ws0, Wn0, b0,
           Wp1, bp1, Ws1, Wn1, b1, Wp2, bp2, Ws2, Wn2, b2):
    src = edge_index[0].astype(jnp.int32)
    dst = edge_index[1].astype(jnp.int32)
    pair, cnts = _prep(src, dst)

    m0 = _tc_pool(inputs, Wp0, bp0)
    n0 = _neigh(m0, pair, cnts)
    h1, m1 = _tc_mid(inputs, n0, Ws0, Wn0, b0, Wp1, bp1)
    n1 = _neigh(m1, pair, cnts)
    enc, h2, m2 = _tc_mid_enc(h1, n1, Ws1, Wn1, b1, Wp2, bp2)
    n2 = _neigh(m2, pair, cnts)
    out = _tc_last(h2, n2, Ws2, Wn2, b2)
    return (out, enc)


# speedup vs baseline: 1.1569x; 1.1569x over previous
import functools

import jax
import jax.numpy as jnp
from jax import lax
from jax.experimental import pallas as pl
from jax.experimental.pallas import tpu as pltpu
from jax.experimental.pallas import tpu_sc as plsc

N = 10000
E = 160000
D = 256

NW = 32
NPW = 313
NPAD = NW * NPW

CHUNK = 16000
NCHUNK = E // CHUNK
SEG = CHUNK
SEGW = NCHUNK * SEG
CROW = 256
G = 64

_MESH = plsc.VectorSubcoreMesh(core_axis_name="c", subcore_axis_name="s")
_SC_PARAMS = pltpu.CompilerParams(needs_layout_passes=False)


def _wid():
    return lax.axis_index("s") * 2 + lax.axis_index("c")


def _prep_body(src_hbm, dst_hbm, pair_hbm, cnt_hbm,
               dstv, srcv, stage_p, cbuf):
    w = _wid()
    lo = w * NPW
    hi = lo + NPW

    def chunk_body(ci, _):
        coff = pl.multiple_of(ci * CHUNK, 128)
        pltpu.sync_copy(dst_hbm.at[pl.ds(coff, CHUNK)], dstv)
        pltpu.sync_copy(src_hbm.at[pl.ds(coff, CHUNK)], srcv)

        def grp(g, fill):
            d16 = dstv[pl.ds(g * 16, 16)]
            s16 = srcv[pl.ds(g * 16, 16)]
            msk = (d16 >= lo) & (d16 < hi)
            ones = jnp.where(msk, jnp.int32(1), jnp.int32(0))
            csum = plsc.cumsum(ones)
            pos2 = 2 * (fill + csum - ones)
            plsc.store_scatter(stage_p, [pos2], s16, mask=msk)
            plsc.store_scatter(stage_p, [pos2 + 1], d16 - lo, mask=msk)
            return fill + csum[15]

        fill = lax.fori_loop(0, CHUNK // 16, grp, 0)
        cbuf[pl.ds(pl.multiple_of(ci * 16, 16), 16)] = (
            jnp.broadcast_to(fill, (16,)).astype(jnp.int32))
        soff = pl.multiple_of(w * 2 * SEGW + ci * 2 * SEG, 128)
        pltpu.sync_copy(stage_p, pair_hbm.at[pl.ds(soff, 2 * SEG)])
        return 0

    lax.fori_loop(0, NCHUNK, chunk_body, 0)
    pltpu.sync_copy(cbuf, cnt_hbm.at[pl.ds(pl.multiple_of(w * CROW, 128),
                                           CROW)])


_prep = pl.kernel(
    _prep_body,
    out_type=(
        jax.ShapeDtypeStruct((NW * 2 * SEGW,), jnp.int32),
        jax.ShapeDtypeStruct((NW * CROW,), jnp.int32),
    ),
    mesh=_MESH,
    scratch_types=[
        pltpu.VMEM((CHUNK,), jnp.int32),
        pltpu.VMEM((CHUNK,), jnp.int32),
        pltpu.VMEM((2 * SEG,), jnp.int32),
        pltpu.VMEM((CROW,), jnp.int32),
    ],
    compiler_params=_SC_PARAMS,
)


def _agg_body(m_hbm, pair_hbm, cnt_hbm, out_hbm,
              acc, rows0, rows1, pbuf0, pbuf1, idxv0, idxv1, edlv0, edlv1,
              cntv, semg0, semg1, semp0, semp1):
    w = _wid()
    rows = (rows0, rows1)
    pbuf = (pbuf0, pbuf1)
    idxv = (idxv0, idxv1)
    edlv = (edlv0, edlv1)
    semg = (semg0, semg1)
    semp = (semp0, semp1)

    def zero(i, _):
        acc[pl.ds(i * 16, 16)] = jnp.zeros((16,), jnp.float32)
        return 0

    lax.fori_loop(0, (NPW + 1) * D // 16, zero, 0)

    pltpu.sync_copy(cnt_hbm.at[pl.ds(pl.multiple_of(w * CROW, 128), CROW)],
                    cntv)

    def seg(ci, _):
        cnt = cntv[pl.ds(pl.multiple_of(ci * 16, 16), 16)][0]
        nblocks = (cnt + G - 1) // G
        segbase = w * 2 * SEGW + ci * 2 * SEG

        def pair_slice(bb):
            off = pl.multiple_of(segbase + bb * 2 * G, 128)
            return pair_hbm.at[pl.ds(off, 2 * G)]

        def startpair(bb, p):
            pltpu.async_copy(pair_slice(bb), pbuf[p], semp[p])

        def preparegather(bb, p):
            pltpu.make_async_copy(pair_slice(bb), pbuf[p], semp[p]).wait()
            iota2 = lax.iota(jnp.int32, 16) * 2
            for j in range(G // 16):
                ev = plsc.load_gather(pbuf[p], [iota2 + (j * 32)])
                od = plsc.load_gather(pbuf[p], [iota2 + (j * 32 + 1)])
                idxv[p][pl.ds(j * 16, 16)] = jnp.clip(ev, 0, N - 1)
                edlv[p][pl.ds(j * 16, 16)] = od
            pltpu.async_copy(m_hbm.at[idxv[p]], rows[p], semg[p])

        def accumulate(bb, p):
            lbase = bb * G
            ne = jnp.minimum(cnt - lbase, G)
            ne_v = jnp.broadcast_to(ne, (16,))
            ngroups = (ne + 15) // 16

            def grp(gi, _):
                gbase = pl.multiple_of(gi * 16, 16)
                dls = edlv[p][pl.ds(gbase, 16)]
                valid = (lax.iota(jnp.int32, 16) + gbase) < ne_v
                dls = jnp.where(valid, jnp.clip(dls, 0, NPW - 1), NPW)
                for k in range(16):
                    boff = dls[k] * D
                    erow = gbase + k
                    for v in range(D // 16):
                        sl = pl.ds(boff + v * 16, 16)
                        acc[sl] = jnp.maximum(
                            acc[sl], rows[p][erow, pl.ds(v * 16, 16)])
                return 0

            lax.fori_loop(0, ngroups, grp, 0)

        @pl.when(nblocks > 0)
        def _prologue():
            startpair(0, 0)
            preparegather(0, 0)

        def outer(ob, _):
            for p in (0, 1):
                bb = ob * 2 + p

                @pl.when((bb >= 1) & (bb <= nblocks))
                def _acc_prev():
                    accumulate(bb - 1, 1 - p)

                @pl.when(bb + 1 < nblocks)
                def _start_next():
                    startpair(bb + 1, 1 - p)

                @pl.when(bb < nblocks)
                def _wait_gather():
                    pltpu.make_async_copy(m_hbm.at[idxv[p]], rows[p],
                                          semg[p]).wait()

                @pl.when(bb + 1 < nblocks)
                def _prep_next():
                    preparegather(bb + 1, 1 - p)
            return 0

        lax.fori_loop(0, (nblocks + 2) // 2, outer, 0)
        return 0

    lax.fori_loop(0, NCHUNK, seg, 0)
    pltpu.sync_copy(acc.at[pl.ds(0, NPW * D)],
                    out_hbm.at[pl.ds(pl.multiple_of(w * NPW * D, 128),
                                     NPW * D)])


_agg = pl.kernel(
    _agg_body,
    out_type=jax.ShapeDtypeStruct((NPAD * D,), jnp.float32),
    mesh=_MESH,
    scratch_types=[
        pltpu.VMEM(((NPW + 1) * D,), jnp.float32),
        pltpu.VMEM((G, D), jnp.float32),
        pltpu.VMEM((G, D), jnp.float32),
        pltpu.VMEM((2 * G,), jnp.int32),
        pltpu.VMEM((2 * G,), jnp.int32),
        pltpu.VMEM((G,), jnp.int32),
        pltpu.VMEM((G,), jnp.int32),
        pltpu.VMEM((G + 16,), jnp.int32),
        pltpu.VMEM((G + 16,), jnp.int32),
        pltpu.VMEM((CROW,), jnp.int32),
        pltpu.SemaphoreType.DMA,
        pltpu.SemaphoreType.DMA,
        pltpu.SemaphoreType.DMA,
        pltpu.SemaphoreType.DMA,
    ],
    compiler_params=_SC_PARAMS,
)


RB = 1000
GRID = N // RB


def _l2norm(x):
    n = jnp.sqrt(jnp.sum(x * x, axis=1, keepdims=True))
    return x / jnp.maximum(n, 1e-12)


def _row_spec():
    return pl.BlockSpec((RB, D), lambda i: (i, 0))


def _full_spec():
    return pl.BlockSpec((D, D), lambda i: (0, 0))


def _bias_spec():
    return pl.BlockSpec((1, D), lambda i: (0, 0))


def _pool_body(h_ref, wp_ref, bp_ref, m_ref):
    m = jnp.dot(h_ref[...], wp_ref[...], preferred_element_type=jnp.float32)
    m_ref[...] = jnp.maximum(m + bp_ref[...], 0.0)


def _tc_pool(h, Wp, bp):
    return pl.pallas_call(
        _pool_body,
        grid=(GRID,),
        in_specs=[_row_spec(), _full_spec(), _bias_spec()],
        out_specs=_row_spec(),
        out_shape=jax.ShapeDtypeStruct((N, D), jnp.float32),
    )(h, Wp, bp.reshape(1, D))


def _mid_body(h_ref, ng_ref, ws_ref, wn_ref, b_ref, wp_ref, bp_ref,
              h1_ref, m1_ref):
    out = jnp.dot(h_ref[...], ws_ref[...], preferred_element_type=jnp.float32)
    out += jnp.dot(ng_ref[...], wn_ref[...], preferred_element_type=jnp.float32)
    out += b_ref[...]
    h1 = _l2norm(jnp.maximum(out, 0.0))
    h1_ref[...] = h1
    m1 = jnp.dot(h1, wp_ref[...], preferred_element_type=jnp.float32)
    m1_ref[...] = jnp.maximum(m1 + bp_ref[...], 0.0)


def _tc_mid(h, ng, Ws, Wn, b, Wp, bp):
    return pl.pallas_call(
        _mid_body,
        grid=(GRID,),
        in_specs=[_row_spec(), _row_spec(), _full_spec(), _full_spec(),
                  _bias_spec(), _full_spec(), _bias_spec()],
        out_specs=[_row_spec(), _row_spec()],
        out_shape=[jax.ShapeDtypeStruct((N, D), jnp.float32),
                   jax.ShapeDtypeStruct((N, D), jnp.float32)],
    )(h, ng, Ws, Wn, b.reshape(1, D), Wp, bp.reshape(1, D))


def _mid_enc_body(h_ref, ng_ref, ws_ref, wn_ref, b_ref, wp_ref, bp_ref,
                  enc_ref, h1_ref, m1_ref):
    out = jnp.dot(h_ref[...], ws_ref[...], preferred_element_type=jnp.float32)
    out += jnp.dot(ng_ref[...], wn_ref[...], preferred_element_type=jnp.float32)
    out += b_ref[...]
    enc_ref[...] = _l2norm(out)
    h1 = _l2norm(jnp.maximum(out, 0.0))
    h1_ref[...] = h1
    m1 = jnp.dot(h1, wp_ref[...], preferred_element_type=jnp.float32)
    m1_ref[...] = jnp.maximum(m1 + bp_ref[...], 0.0)


def _tc_mid_enc(h, ng, Ws, Wn, b, Wp, bp):
    return pl.pallas_call(
        _mid_enc_body,
        grid=(GRID,),
        in_specs=[_row_spec(), _row_spec(), _full_spec(), _full_spec(),
                  _bias_spec(), _full_spec(), _bias_spec()],
        out_specs=[_row_spec(), _row_spec(), _row_spec()],
        out_shape=[jax.ShapeDtypeStruct((N, D), jnp.float32),
                   jax.ShapeDtypeStruct((N, D), jnp.float32),
                   jax.ShapeDtypeStruct((N, D), jnp.float32)],
    )(h, ng, Ws, Wn, b.reshape(1, D), Wp, bp.reshape(1, D))


def _last_body(h_ref, ng_ref, ws_ref, wn_ref, b_ref, out_ref):
    out = jnp.dot(h_ref[...], ws_ref[...], preferred_element_type=jnp.float32)
    out += jnp.dot(ng_ref[...], wn_ref[...], preferred_element_type=jnp.float32)
    out_ref[...] = out + b_ref[...]


def _tc_last(h, ng, Ws, Wn, b):
    return pl.pallas_call(
        _last_body,
        grid=(GRID,),
        in_specs=[_row_spec(), _row_spec(), _full_spec(), _full_spec(),
                  _bias_spec()],
        out_specs=_row_spec(),
        out_shape=jax.ShapeDtypeStruct((N, D), jnp.float32),
    )(h, ng, Ws, Wn, b.reshape(1, D))


def _neigh(m, pair, cnts):
    flat = _agg(m, pair, cnts)
    return flat.reshape(NPAD, D)[:N]


def kernel(inputs, edge_index, Wp0, bp0, Ws0, Wn0, b0,
           Wp1, bp1, Ws1, Wn1, b1, Wp2, bp2, Ws2, Wn2, b2):
    src = edge_index[0].astype(jnp.int32)
    dst = edge_index[1].astype(jnp.int32)
    pair, cnts = _prep(src, dst)

    m0 = _tc_pool(inputs, Wp0, bp0)
    n0 = _neigh(m0, pair, cnts)
    h1, m1 = _tc_mid(inputs, n0, Ws0, Wn0, b0, Wp1, bp1)
    n1 = _neigh(m1, pair, cnts)
    enc, h2, m2 = _tc_mid_enc(h1, n1, Ws1, Wn1, b1, Wp2, bp2)
    n2 = _neigh(m2, pair, cnts)
    out = _tc_last(h2, n2, Ws2, Wn2, b2)
    return (out, enc)
